# baseline (speedup 1.0000x reference)
import jax
import jax.numpy as jnp
from jax.experimental import pallas as pl

NBIRD = 50000
NCOLOR = 16
H = 64
BB = 1000
NBLK = NBIRD // BB
F32 = jnp.float32
_EDGE_DT = jnp.float32


def _dot(a, b):
    return jnp.dot(a, b, preferred_element_type=F32)


def _csum_accum(csum_ref, part):
    pid = pl.program_id(0)

    @pl.when(pid == 0)
    def _():
        csum_ref[:] = part

    @pl.when(pid > 0)
    def _():
        csum_ref[:] = csum_ref[:] + part


def _layer0_body(probs_ref, wnT_ref, bn_ref, aT_ref, u_ref, base_ref,
                 e2T_ref, eb2_ref, n1aT_ref, n1bT_ref, nb1_ref, n2T_ref,
                 nb2_ref, e_out_ref, x_out_ref, csum_ref):
    p = probs_ref[:]
    xb = _dot(p, wnT_ref[:]) + bn_ref[:]
    ba = _dot(xb, aT_ref[:])
    u = u_ref[:]
    e2T = e2T_ref[:]
    eb2 = eb2_ref[:]
    aggr = None
    parts = []
    for c in range(NCOLOR):
        pre = ba + p[:, c:c + 1] * u + base_ref[c:c + 1, :]
        e_new = _dot(jnp.maximum(pre, 0.0), e2T) + eb2
        e_out_ref[c] = e_new.astype(e_out_ref.dtype)
        aggr = e_new if aggr is None else aggr + e_new
        parts.append(jnp.sum(e_new, axis=0, keepdims=True))
    part = jnp.concatenate(parts, axis=0)
    h2 = jnp.maximum(
        _dot(xb, n1aT_ref[:]) + _dot(aggr, n1bT_ref[:]) + nb1_ref[:], 0.0)
    x_out_ref[:] = _dot(h2, n2T_ref[:]) + nb2_ref[:]
    _csum_accum(csum_ref, part)


def _mid_body(e_ref, x_ref, aT_ref, cT_ref, base_ref, e2T_ref, eb2_ref,
              n1aT_ref, n1bT_ref, nb1_ref, n2T_ref, nb2_ref,
              e_out_ref, x_out_ref, csum_ref):
    x = x_ref[:]
    ba = _dot(x, aT_ref[:])
    cT = cT_ref[:]
    e2T = e2T_ref[:]
    eb2 = eb2_ref[:]
    aggr = None
    parts = []
    for c in range(NCOLOR):
        pre = ba + _dot(e_ref[c].astype(F32), cT) + base_ref[c:c + 1, :]
        e_new = _dot(jnp.maximum(pre, 0.0), e2T) + eb2
        e_out_ref[c] = e_new.astype(e_out_ref.dtype)
        aggr = e_new if aggr is None else aggr + e_new
        parts.append(jnp.sum(e_new, axis=0, keepdims=True))
    part = jnp.concatenate(parts, axis=0)
    h2 = jnp.maximum(
        _dot(x, n1aT_ref[:]) + _dot(aggr, n1bT_ref[:]) + nb1_ref[:], 0.0)
    x_out_ref[:] = _dot(h2, n2T_ref[:]) + nb2_ref[:]
    _csum_accum(csum_ref, part)


def _last_body(e_ref, x_ref, probs_ref, aT_ref, cT_ref, base_ref, e2T_ref,
               eb2_ref, n1aT_ref, n1bT_ref, nb1_ref, n2T_ref, nb2_ref,
               wcT_ref, bc_ref, out_ref):
    x = x_ref[:]
    ba = _dot(x, aT_ref[:])
    cT = cT_ref[:]
    e2T = e2T_ref[:]
    eb2 = eb2_ref[:]
    aggr = None
    for c in range(NCOLOR):
        pre = ba + _dot(e_ref[c].astype(F32), cT) + base_ref[c:c + 1, :]
        e_new = _dot(jnp.maximum(pre, 0.0), e2T) + eb2
        aggr = e_new if aggr is None else aggr + e_new
    h2 = jnp.maximum(
        _dot(x, n1aT_ref[:]) + _dot(aggr, n1bT_ref[:]) + nb1_ref[:], 0.0)
    xn = _dot(h2, n2T_ref[:]) + nb2_ref[:]
    scores = _dot(xn, wcT_ref[:]) + bc_ref[:]
    out_ref[:] = scores * probs_ref[:]


def _full(shape):
    return pl.BlockSpec(shape, lambda i: tuple(0 for _ in shape))


def kernel(probs, Wn, bn, We, be, eW1, eb1, eW2, eb2, nW1, nb1, nW2, nb2,
           Wc, bc):
    f = lambda a: a.astype(F32)
    probs = f(probs)
    wnT = f(Wn).T
    x_color = wnT + f(bn)[None, :]
    A = [f(eW1[l][:, :H]).T for l in range(3)]
    Bm = [f(eW1[l][:, H:2 * H]).T for l in range(3)]
    Cm = [f(eW1[l][:, 2 * H:]).T for l in range(3)]
    E2 = [f(eW2[l]).T for l in range(3)]
    N1a = [f(nW1[l][:, :H]).T for l in range(3)]
    N1b = [f(nW1[l][:, H:]).T for l in range(3)]
    N2 = [f(nW2[l]).T for l in range(3)]
    eb1_ = [f(eb1[l])[None, :] for l in range(3)]
    eb2_ = [f(eb2[l])[None, :] for l in range(3)]
    nb1_ = [f(nb1[l])[None, :] for l in range(3)]
    nb2_ = [f(nb2[l])[None, :] for l in range(3)]
    bn_r = f(bn)[None, :]
    u0 = (f(We)[:, 0] @ Cm[0])[None, :]
    v0 = (f(be) @ Cm[0])[None, :]

    def edge_base(l, xc):
        b = xc @ Bm[l] + eb1_[l]
        if l == 0:
            b = b + v0
        return b

    def color_update(l, xc, aggr_c):
        h2 = jnp.maximum(xc @ N1a[l] + aggr_c @ N1b[l] + nb1_[l], 0.0)
        return h2 @ N2[l] + nb2_[l]

    wspecs = [_full((H, H)), _full((H, H)), _full((NCOLOR, H)),
              _full((H, H)), _full((1, H)), _full((H, H)), _full((H, H)),
              _full((1, H)), _full((H, H)), _full((1, H))]
    e_spec = pl.BlockSpec((NCOLOR, BB, H), lambda i: (0, i, 0))
    x_spec = pl.BlockSpec((BB, H), lambda i: (i, 0))
    p_spec = pl.BlockSpec((BB, NCOLOR), lambda i: (i, 0))
    csum_spec = pl.BlockSpec((NCOLOR, H), lambda i: (0, 0))
    e_shape = jax.ShapeDtypeStruct((NCOLOR, NBIRD, H), _EDGE_DT)
    x_shape = jax.ShapeDtypeStruct((NBIRD, H), F32)
    csum_shape = jax.ShapeDtypeStruct((NCOLOR, H), F32)

    e1, x1, csum = pl.pallas_call(
        _layer0_body,
        grid=(NBLK,),
        in_specs=[p_spec, _full((NCOLOR, H)), _full((1, H)), _full((H, H)),
                  _full((1, H)), _full((NCOLOR, H)), _full((H, H)),
                  _full((1, H)), _full((H, H)), _full((H, H)), _full((1, H)),
                  _full((H, H)), _full((1, H))],
        out_specs=[e_spec, x_spec, csum_spec],
        out_shape=[e_shape, x_shape, csum_shape],
    )(probs, wnT, bn_r, A[0], u0, edge_base(0, x_color), E2[0], eb2_[0],
      N1a[0], N1b[0], nb1_[0], N2[0], nb2_[0])
    x_color = color_update(0, x_color, csum)

    e2, x2, csum = pl.pallas_call(
        _mid_body,
        grid=(NBLK,),
        in_specs=[e_spec, x_spec] + wspecs,
        out_specs=[e_spec, x_spec, csum_spec],
        out_shape=[e_shape, x_shape, csum_shape],
    )(e1, x1, A[1], Cm[1], edge_base(1, x_color), E2[1], eb2_[1],
      N1a[1], N1b[1], nb1_[1], N2[1], nb2_[1])
    x_color = color_update(1, x_color, csum)

    out = pl.pallas_call(
        _last_body,
        grid=(NBLK,),
        in_specs=[e_spec, x_spec, p_spec] + wspecs + [_full((H, NCOLOR)),
                                                      _full((1, NCOLOR))],
        out_specs=p_spec,
        out_shape=jax.ShapeDtypeStruct((NBIRD, NCOLOR), F32),
    )(e2, x2, probs, A[2], Cm[2], edge_base(2, x_color), E2[2], eb2_[2],
      N1a[2], N1b[2], nb1_[2], N2[2], nb2_[2], f(Wc).T, f(bc)[None, :])
    return out


# speedup vs baseline: 1.0298x; 1.0298x over previous
import jax
import jax.numpy as jnp
from jax.experimental import pallas as pl

NBIRD = 50000
NCOLOR = 16
H = 64
BB = 1000
NBLK = NBIRD // BB
F32 = jnp.float32
BF16 = jnp.bfloat16
_EDGE_DT = jnp.bfloat16


def _dot(a, b):
    return jnp.dot(a, b, preferred_element_type=F32)


def _csum_accum(csum_ref, part):
    pid = pl.program_id(0)

    @pl.when(pid == 0)
    def _():
        csum_ref[:] = part

    @pl.when(pid > 0)
    def _():
        csum_ref[:] = csum_ref[:] + part


def _layer0_body(probs_ref, wnT_ref, bn_ref, aT_ref, u_ref, base_ref,
                 e2T_ref, eb2_ref, n1aT_ref, n1bT_ref, nb1_ref, n2T_ref,
                 nb2_ref, e_out_ref, x_out_ref, csum_ref):
    p = probs_ref[:]
    xb = _dot(p, wnT_ref[:]) + bn_ref[:]
    ba = _dot(xb, aT_ref[:])
    u = u_ref[:]
    e2T = e2T_ref[:]
    eb2 = eb2_ref[:]
    aggr = None
    parts = []
    for c in range(NCOLOR):
        pre = ba + p[:, c:c + 1] * u + base_ref[c:c + 1, :]
        e_new = _dot(jnp.maximum(pre, 0.0).astype(BF16), e2T) + eb2
        e_out_ref[c] = e_new.astype(e_out_ref.dtype)
        aggr = e_new if aggr is None else aggr + e_new
        parts.append(jnp.sum(e_new, axis=0, keepdims=True))
    part = jnp.concatenate(parts, axis=0)
    h2 = jnp.maximum(
        _dot(xb, n1aT_ref[:]) + _dot(aggr, n1bT_ref[:]) + nb1_ref[:], 0.0)
    x_out_ref[:] = _dot(h2, n2T_ref[:]) + nb2_ref[:]
    _csum_accum(csum_ref, part)


def _mid_body(e_ref, x_ref, aT_ref, cT_ref, base_ref, e2T_ref, eb2_ref,
              n1aT_ref, n1bT_ref, nb1_ref, n2T_ref, nb2_ref,
              e_out_ref, x_out_ref, csum_ref):
    x = x_ref[:]
    ba = _dot(x, aT_ref[:])
    cT = cT_ref[:]
    e2T = e2T_ref[:]
    eb2 = eb2_ref[:]
    aggr = None
    parts = []
    for c in range(NCOLOR):
        pre = ba + _dot(e_ref[c], cT) + base_ref[c:c + 1, :]
        e_new = _dot(jnp.maximum(pre, 0.0).astype(BF16), e2T) + eb2
        e_out_ref[c] = e_new.astype(e_out_ref.dtype)
        aggr = e_new if aggr is None else aggr + e_new
        parts.append(jnp.sum(e_new, axis=0, keepdims=True))
    part = jnp.concatenate(parts, axis=0)
    h2 = jnp.maximum(
        _dot(x, n1aT_ref[:]) + _dot(aggr, n1bT_ref[:]) + nb1_ref[:], 0.0)
    x_out_ref[:] = _dot(h2, n2T_ref[:]) + nb2_ref[:]
    _csum_accum(csum_ref, part)


def _last_body(e_ref, x_ref, probs_ref, aT_ref, cT_ref, base_ref, e2T_ref,
               eb2_ref, n1aT_ref, n1bT_ref, nb1_ref, n2T_ref, nb2_ref,
               wcT_ref, bc_ref, out_ref):
    x = x_ref[:]
    ba = _dot(x, aT_ref[:])
    cT = cT_ref[:]
    e2T = e2T_ref[:]
    eb2 = eb2_ref[:]
    aggr = None
    for c in range(NCOLOR):
        pre = ba + _dot(e_ref[c], cT) + base_ref[c:c + 1, :]
        e_new = _dot(jnp.maximum(pre, 0.0).astype(BF16), e2T) + eb2
        aggr = e_new if aggr is None else aggr + e_new
    h2 = jnp.maximum(
        _dot(x, n1aT_ref[:]) + _dot(aggr, n1bT_ref[:]) + nb1_ref[:], 0.0)
    xn = _dot(h2, n2T_ref[:]) + nb2_ref[:]
    scores = _dot(xn, wcT_ref[:]) + bc_ref[:]
    out_ref[:] = scores * probs_ref[:]


def _full(shape):
    return pl.BlockSpec(shape, lambda i: tuple(0 for _ in shape))


def kernel(probs, Wn, bn, We, be, eW1, eb1, eW2, eb2, nW1, nb1, nW2, nb2,
           Wc, bc):
    f = lambda a: a.astype(F32)
    probs = f(probs)
    wnT = f(Wn).T
    x_color = wnT + f(bn)[None, :]
    A = [f(eW1[l][:, :H]).T for l in range(3)]
    Bm = [f(eW1[l][:, H:2 * H]).T for l in range(3)]
    Cm = [f(eW1[l][:, 2 * H:]).T for l in range(3)]
    E2 = [f(eW2[l]).T for l in range(3)]
    N1a = [f(nW1[l][:, :H]).T for l in range(3)]
    N1b = [f(nW1[l][:, H:]).T for l in range(3)]
    N2 = [f(nW2[l]).T for l in range(3)]
    eb1_ = [f(eb1[l])[None, :] for l in range(3)]
    eb2_ = [f(eb2[l])[None, :] for l in range(3)]
    nb1_ = [f(nb1[l])[None, :] for l in range(3)]
    nb2_ = [f(nb2[l])[None, :] for l in range(3)]
    bn_r = f(bn)[None, :]
    u0 = (f(We)[:, 0] @ Cm[0])[None, :]
    v0 = (f(be) @ Cm[0])[None, :]

    def edge_base(l, xc):
        b = xc @ Bm[l] + eb1_[l]
        if l == 0:
            b = b + v0
        return b

    def color_update(l, xc, aggr_c):
        h2 = jnp.maximum(xc @ N1a[l] + aggr_c @ N1b[l] + nb1_[l], 0.0)
        return h2 @ N2[l] + nb2_[l]

    wspecs = [_full((H, H)), _full((H, H)), _full((NCOLOR, H)),
              _full((H, H)), _full((1, H)), _full((H, H)), _full((H, H)),
              _full((1, H)), _full((H, H)), _full((1, H))]
    e_spec = pl.BlockSpec((NCOLOR, BB, H), lambda i: (0, i, 0))
    x_spec = pl.BlockSpec((BB, H), lambda i: (i, 0))
    p_spec = pl.BlockSpec((BB, NCOLOR), lambda i: (i, 0))
    csum_spec = pl.BlockSpec((NCOLOR, H), lambda i: (0, 0))
    e_shape = jax.ShapeDtypeStruct((NCOLOR, NBIRD, H), _EDGE_DT)
    x_shape = jax.ShapeDtypeStruct((NBIRD, H), F32)
    csum_shape = jax.ShapeDtypeStruct((NCOLOR, H), F32)

    e1, x1, csum = pl.pallas_call(
        _layer0_body,
        grid=(NBLK,),
        in_specs=[p_spec, _full((NCOLOR, H)), _full((1, H)), _full((H, H)),
                  _full((1, H)), _full((NCOLOR, H)), _full((H, H)),
                  _full((1, H)), _full((H, H)), _full((H, H)), _full((1, H)),
                  _full((H, H)), _full((1, H))],
        out_specs=[e_spec, x_spec, csum_spec],
        out_shape=[e_shape, x_shape, csum_shape],
    )(probs, wnT, bn_r, A[0], u0, edge_base(0, x_color), E2[0].astype(BF16), eb2_[0],
      N1a[0], N1b[0], nb1_[0], N2[0], nb2_[0])
    x_color = color_update(0, x_color, csum)

    e2, x2, csum = pl.pallas_call(
        _mid_body,
        grid=(NBLK,),
        in_specs=[e_spec, x_spec] + wspecs,
        out_specs=[e_spec, x_spec, csum_spec],
        out_shape=[e_shape, x_shape, csum_shape],
    )(e1, x1, A[1], Cm[1].astype(BF16), edge_base(1, x_color), E2[1].astype(BF16), eb2_[1],
      N1a[1], N1b[1], nb1_[1], N2[1], nb2_[1])
    x_color = color_update(1, x_color, csum)

    out = pl.pallas_call(
        _last_body,
        grid=(NBLK,),
        in_specs=[e_spec, x_spec, p_spec] + wspecs + [_full((H, NCOLOR)),
                                                      _full((1, NCOLOR))],
        out_specs=p_spec,
        out_shape=jax.ShapeDtypeStruct((NBIRD, NCOLOR), F32),
    )(e2, x2, probs, A[2], Cm[2].astype(BF16), edge_base(2, x_color), E2[2].astype(BF16), eb2_[2],
      N1a[2], N1b[2], nb1_[2], N2[2], nb2_[2], f(Wc).T, f(bc)[None, :])
    return out


# speedup vs baseline: 1.4636x; 1.4212x over previous
import jax
import jax.numpy as jnp
from jax.experimental import pallas as pl

NBIRD = 50000
NCOLOR = 16
NP = NCOLOR // 2
H = 64
H2 = 2 * H
BB = 1000
NBLK = NBIRD // BB
F32 = jnp.float32
BF16 = jnp.bfloat16


def _dot(a, b):
    return jnp.dot(a, b, preferred_element_type=F32)


def _csum_accum(csum_ref, part):
    pid = pl.program_id(0)

    @pl.when(pid == 0)
    def _():
        csum_ref[:] = part

    @pl.when(pid > 0)
    def _():
        csum_ref[:] = csum_ref[:] + part


def _layer0_body(probs_ref, wnT_ref, bn_ref, aa_ref, su_ref, base2_ref,
                 e2bd_ref, n1aT_ref, n1bS_ref, nb1_ref, n2T_ref, nb2_ref,
                 e_out_ref, x_out_ref, csum_ref):
    p = probs_ref[:]
    xb = _dot(p, wnT_ref[:]) + bn_ref[:]
    ba2 = _dot(xb, aa_ref[:])
    pb = p.astype(BF16)
    e2bd = e2bd_ref[:]
    aggr2 = None
    parts = []
    for g in range(NP):
        pre = _dot(pb, su_ref[g]) + ba2 + base2_ref[g:g + 1, :]
        en = _dot(jnp.maximum(pre, 0.0).astype(BF16), e2bd)
        e_out_ref[g] = en.astype(e_out_ref.dtype)
        aggr2 = en if aggr2 is None else aggr2 + en
        parts.append(jnp.sum(en, axis=0, keepdims=True))
    part = jnp.concatenate(parts, axis=0)
    h2 = jnp.maximum(
        _dot(xb, n1aT_ref[:]) + _dot(aggr2, n1bS_ref[:]) + nb1_ref[:], 0.0)
    x_out_ref[:] = _dot(h2, n2T_ref[:]) + nb2_ref[:]
    _csum_accum(csum_ref, part)


def _mid_body(e_ref, x_ref, aa_ref, cbd_ref, base2_ref, e2bd_ref,
              n1aT_ref, n1bS_ref, nb1_ref, n2T_ref, nb2_ref,
              e_out_ref, x_out_ref, csum_ref):
    x = x_ref[:]
    ba2 = _dot(x, aa_ref[:])
    cbd = cbd_ref[:]
    e2bd = e2bd_ref[:]
    aggr2 = None
    parts = []
    for g in range(NP):
        pre = _dot(e_ref[g], cbd) + ba2 + base2_ref[g:g + 1, :]
        en = _dot(jnp.maximum(pre, 0.0).astype(BF16), e2bd)
        e_out_ref[g] = en.astype(e_out_ref.dtype)
        aggr2 = en if aggr2 is None else aggr2 + en
        parts.append(jnp.sum(en, axis=0, keepdims=True))
    part = jnp.concatenate(parts, axis=0)
    h2 = jnp.maximum(
        _dot(x, n1aT_ref[:]) + _dot(aggr2, n1bS_ref[:]) + nb1_ref[:], 0.0)
    x_out_ref[:] = _dot(h2, n2T_ref[:]) + nb2_ref[:]
    _csum_accum(csum_ref, part)


def _last_body(e_ref, x_ref, probs_ref, aa_ref, cbd_ref, base2_ref,
               e2bd_ref, n1aT_ref, n1bS_ref, nb1_ref, n2T_ref, nb2_ref,
               wcT_ref, bc_ref, out_ref):
    x = x_ref[:]
    ba2 = _dot(x, aa_ref[:])
    cbd = cbd_ref[:]
    e2bd = e2bd_ref[:]
    aggr2 = None
    for g in range(NP):
        pre = _dot(e_ref[g], cbd) + ba2 + base2_ref[g:g + 1, :]
        en = _dot(jnp.maximum(pre, 0.0).astype(BF16), e2bd)
        aggr2 = en if aggr2 is None else aggr2 + en
    h2 = jnp.maximum(
        _dot(x, n1aT_ref[:]) + _dot(aggr2, n1bS_ref[:]) + nb1_ref[:], 0.0)
    xn = _dot(h2, n2T_ref[:]) + nb2_ref[:]
    scores = _dot(xn, wcT_ref[:]) + bc_ref[:]
    out_ref[:] = scores * probs_ref[:]


def _full(shape):
    return pl.BlockSpec(shape, lambda i: tuple(0 for _ in shape))


def kernel(probs, Wn, bn, We, be, eW1, eb1, eW2, eb2, nW1, nb1, nW2, nb2,
           Wc, bc):
    f = lambda a: a.astype(F32)
    probs = f(probs)
    wnT = f(Wn).T
    x_color = wnT + f(bn)[None, :]
    A = [f(eW1[l][:, :H]).T for l in range(3)]
    Bm = [f(eW1[l][:, H:2 * H]).T for l in range(3)]
    Cm = [f(eW1[l][:, 2 * H:]).T for l in range(3)]
    E2 = [f(eW2[l]).T for l in range(3)]
    N1a = [f(nW1[l][:, :H]).T for l in range(3)]
    N1b = [f(nW1[l][:, H:]).T for l in range(3)]
    N2 = [f(nW2[l]).T for l in range(3)]
    eb1_ = [f(eb1[l])[None, :] for l in range(3)]
    eb2_ = [f(eb2[l])[None, :] for l in range(3)]
    nb1_ = [f(nb1[l])[None, :] for l in range(3)]
    nb2_ = [f(nb2[l])[None, :] for l in range(3)]
    bn_r = f(bn)[None, :]
    u0 = (f(We)[:, 0] @ Cm[0])[None, :]
    v0 = (f(be) @ Cm[0])[None, :]

    def bd(w):
        z = jnp.zeros_like(w)
        top = jnp.concatenate([w, z], axis=1)
        bot = jnp.concatenate([z, w], axis=1)
        return jnp.concatenate([top, bot], axis=0)

    AA = [jnp.concatenate([A[l], A[l]], axis=1) for l in range(3)]
    CBD = [bd(Cm[l]).astype(BF16) for l in range(3)]
    E2BD = [bd(E2[l]).astype(BF16) for l in range(3)]
    N1bS = [jnp.concatenate([N1b[l], N1b[l]], axis=0) for l in range(3)]
    nb1f = [nb1_[l] + (NCOLOR * eb2_[l]) @ N1b[l] for l in range(3)]

    def base2(l, xc):
        b = xc @ Bm[l] + eb1_[l]
        if l == 0:
            b = b + v0
        else:
            b = b + eb2_[l - 1] @ Cm[l]
        return b.reshape(NP, H2)

    def color_update(l, xc, csum):
        aggr_c = csum.reshape(NCOLOR, H) + NBIRD * eb2_[l]
        h2 = jnp.maximum(xc @ N1a[l] + aggr_c @ N1b[l] + nb1_[l], 0.0)
        return h2 @ N2[l] + nb2_[l]

    eyeN = jnp.eye(NCOLOR, dtype=F32)
    su = jnp.stack([
        jnp.concatenate([eyeN[:, 2 * g:2 * g + 1] @ u0,
                         eyeN[:, 2 * g + 1:2 * g + 2] @ u0], axis=1)
        for g in range(NP)
    ]).astype(BF16)

    wspecs = [_full((H, H2)), _full((H2, H2)), _full((NP, H2)),
              _full((H2, H2)), _full((H, H)), _full((H2, H)),
              _full((1, H)), _full((H, H)), _full((1, H))]
    e_spec = pl.BlockSpec((NP, BB, H2), lambda i: (0, i, 0))
    x_spec = pl.BlockSpec((BB, H), lambda i: (i, 0))
    p_spec = pl.BlockSpec((BB, NCOLOR), lambda i: (i, 0))
    csum_spec = pl.BlockSpec((NP, H2), lambda i: (0, 0))
    e_shape = jax.ShapeDtypeStruct((NP, NBIRD, H2), BF16)
    x_shape = jax.ShapeDtypeStruct((NBIRD, H), F32)
    csum_shape = jax.ShapeDtypeStruct((NP, H2), F32)

    e1, x1, csum = pl.pallas_call(
        _layer0_body,
        grid=(NBLK,),
        in_specs=[p_spec, _full((NCOLOR, H)), _full((1, H)),
                  _full((H, H2)), _full((NP, NCOLOR, H2)), _full((NP, H2)),
                  _full((H2, H2)), _full((H, H)), _full((H2, H)),
                  _full((1, H)), _full((H, H)), _full((1, H))],
        out_specs=[e_spec, x_spec, csum_spec],
        out_shape=[e_shape, x_shape, csum_shape],
    )(probs, wnT, bn_r, AA[0], su, base2(0, x_color), E2BD[0],
      N1a[0], N1bS[0], nb1f[0], N2[0], nb2_[0])
    x_color = color_update(0, x_color, csum)

    e2, x2, csum = pl.pallas_call(
        _mid_body,
        grid=(NBLK,),
        in_specs=[e_spec, x_spec] + wspecs,
        out_specs=[e_spec, x_spec, csum_spec],
        out_shape=[e_shape, x_shape, csum_shape],
    )(e1, x1, AA[1], CBD[1], base2(1, x_color), E2BD[1],
      N1a[1], N1bS[1], nb1f[1], N2[1], nb2_[1])
    x_color = color_update(1, x_color, csum)

    out = pl.pallas_call(
        _last_body,
        grid=(NBLK,),
        in_specs=[e_spec, x_spec, p_spec] + wspecs + [_full((H, NCOLOR)),
                                                      _full((1, NCOLOR))],
        out_specs=p_spec,
        out_shape=jax.ShapeDtypeStruct((NBIRD, NCOLOR), F32),
    )(e2, x2, probs, AA[2], CBD[2], base2(2, x_color), E2BD[2],
      N1a[2], N1bS[2], nb1f[2], N2[2], nb2_[2], f(Wc).T, f(bc)[None, :])
    return out


# speedup vs baseline: 1.7377x; 1.1873x over previous
import jax
import jax.numpy as jnp
from jax.experimental import pallas as pl

NBIRD = 50000
NCOLOR = 16
NP = NCOLOR // 2
H = 64
H2 = 2 * H
BB = 1000
NBLK = NBIRD // BB
F32 = jnp.float32
BF16 = jnp.bfloat16


def _dot(a, b):
    return jnp.dot(a, b, preferred_element_type=F32)


def _csum_accum(csum_ref, part):
    pid = pl.program_id(0)

    @pl.when(pid == 0)
    def _():
        csum_ref[:] = part

    @pl.when(pid > 0)
    def _():
        csum_ref[:] = csum_ref[:] + part


def _layer0_body(probs_ref, wnT_ref, bn_ref, aa_ref, su_ref, base2_ref,
                 e2bd_ref, n1aT_ref, n1bS_ref, nb1_ref, n2T_ref, nb2_ref,
                 e_out_ref, x_out_ref, csum_ref):
    p = probs_ref[:]
    xb = _dot(p, wnT_ref[:]) + bn_ref[:]
    ba2 = _dot(xb, aa_ref[:])
    pb = p.astype(BF16)
    e2bd = e2bd_ref[:]
    pu = jnp.concatenate([_dot(pb, su_ref[g]) for g in range(NP)],
                         axis=0).reshape(NP, BB, H2)
    pre = pu + ba2[None, :, :] + base2_ref[:][:, None, :]
    h = jnp.maximum(pre, 0.0).astype(BF16).reshape(NP * BB, H2)
    en = _dot(h, e2bd_ref[:]).reshape(NP, BB, H2)
    e_out_ref[:] = en.astype(e_out_ref.dtype)
    aggr2 = jnp.sum(en, axis=0)
    part = jnp.sum(en, axis=1)
    h2 = jnp.maximum(
        _dot(xb, n1aT_ref[:]) + _dot(aggr2, n1bS_ref[:]) + nb1_ref[:], 0.0)
    x_out_ref[:] = _dot(h2, n2T_ref[:]) + nb2_ref[:]
    _csum_accum(csum_ref, part)


def _mid_body(e_ref, x_ref, aa_ref, cbd_ref, base2_ref, e2bd_ref,
              n1aT_ref, n1bS_ref, nb1_ref, n2T_ref, nb2_ref,
              e_out_ref, x_out_ref, csum_ref):
    x = x_ref[:]
    ba2 = _dot(x, aa_ref[:])
    cbd = cbd_ref[:]
    e2bd = e2bd_ref[:]
    ec = _dot(e_ref[:].reshape(NP * BB, H2), cbd).reshape(NP, BB, H2)
    pre = ec + ba2[None, :, :] + base2_ref[:][:, None, :]
    h = jnp.maximum(pre, 0.0).astype(BF16).reshape(NP * BB, H2)
    en = _dot(h, e2bd).reshape(NP, BB, H2)
    e_out_ref[:] = en.astype(e_out_ref.dtype)
    aggr2 = jnp.sum(en, axis=0)
    part = jnp.sum(en, axis=1)
    h2 = jnp.maximum(
        _dot(x, n1aT_ref[:]) + _dot(aggr2, n1bS_ref[:]) + nb1_ref[:], 0.0)
    x_out_ref[:] = _dot(h2, n2T_ref[:]) + nb2_ref[:]
    _csum_accum(csum_ref, part)


def _last_body(e_ref, x_ref, probs_ref, aa_ref, cbd_ref, base2_ref,
               e2bd_ref, n1aT_ref, n1bS_ref, nb1_ref, n2T_ref, nb2_ref,
               wcT_ref, bc_ref, out_ref):
    x = x_ref[:]
    ba2 = _dot(x, aa_ref[:])
    cbd = cbd_ref[:]
    e2bd = e2bd_ref[:]
    ec = _dot(e_ref[:].reshape(NP * BB, H2), cbd).reshape(NP, BB, H2)
    pre = ec + ba2[None, :, :] + base2_ref[:][:, None, :]
    h = jnp.maximum(pre, 0.0).astype(BF16).reshape(NP * BB, H2)
    en = _dot(h, e2bd).reshape(NP, BB, H2)
    aggr2 = jnp.sum(en, axis=0)
    h2 = jnp.maximum(
        _dot(x, n1aT_ref[:]) + _dot(aggr2, n1bS_ref[:]) + nb1_ref[:], 0.0)
    xn = _dot(h2, n2T_ref[:]) + nb2_ref[:]
    scores = _dot(xn, wcT_ref[:]) + bc_ref[:]
    out_ref[:] = scores * probs_ref[:]


def _full(shape):
    return pl.BlockSpec(shape, lambda i: tuple(0 for _ in shape))


def kernel(probs, Wn, bn, We, be, eW1, eb1, eW2, eb2, nW1, nb1, nW2, nb2,
           Wc, bc):
    f = lambda a: a.astype(F32)
    probs = f(probs)
    wnT = f(Wn).T
    x_color = wnT + f(bn)[None, :]
    A = [f(eW1[l][:, :H]).T for l in range(3)]
    Bm = [f(eW1[l][:, H:2 * H]).T for l in range(3)]
    Cm = [f(eW1[l][:, 2 * H:]).T for l in range(3)]
    E2 = [f(eW2[l]).T for l in range(3)]
    N1a = [f(nW1[l][:, :H]).T for l in range(3)]
    N1b = [f(nW1[l][:, H:]).T for l in range(3)]
    N2 = [f(nW2[l]).T for l in range(3)]
    eb1_ = [f(eb1[l])[None, :] for l in range(3)]
    eb2_ = [f(eb2[l])[None, :] for l in range(3)]
    nb1_ = [f(nb1[l])[None, :] for l in range(3)]
    nb2_ = [f(nb2[l])[None, :] for l in range(3)]
    bn_r = f(bn)[None, :]
    u0 = (f(We)[:, 0] @ Cm[0])[None, :]
    v0 = (f(be) @ Cm[0])[None, :]

    def bd(w):
        z = jnp.zeros_like(w)
        top = jnp.concatenate([w, z], axis=1)
        bot = jnp.concatenate([z, w], axis=1)
        return jnp.concatenate([top, bot], axis=0)

    AA = [jnp.concatenate([A[l], A[l]], axis=1) for l in range(3)]
    CBD = [bd(Cm[l]).astype(BF16) for l in range(3)]
    E2BD = [bd(E2[l]).astype(BF16) for l in range(3)]
    N1bS = [jnp.concatenate([N1b[l], N1b[l]], axis=0) for l in range(3)]
    nb1f = [nb1_[l] + (NCOLOR * eb2_[l]) @ N1b[l] for l in range(3)]

    def base2(l, xc):
        b = xc @ Bm[l] + eb1_[l]
        if l == 0:
            b = b + v0
        else:
            b = b + eb2_[l - 1] @ Cm[l]
        return b.reshape(NP, H2)

    def color_update(l, xc, csum):
        aggr_c = csum.reshape(NCOLOR, H) + NBIRD * eb2_[l]
        h2 = jnp.maximum(xc @ N1a[l] + aggr_c @ N1b[l] + nb1_[l], 0.0)
        return h2 @ N2[l] + nb2_[l]

    eyeN = jnp.eye(NCOLOR, dtype=F32)
    su = jnp.stack([
        jnp.concatenate([eyeN[:, 2 * g:2 * g + 1] @ u0,
                         eyeN[:, 2 * g + 1:2 * g + 2] @ u0], axis=1)
        for g in range(NP)
    ]).astype(BF16)

    wspecs = [_full((H, H2)), _full((H2, H2)), _full((NP, H2)),
              _full((H2, H2)), _full((H, H)), _full((H2, H)),
              _full((1, H)), _full((H, H)), _full((1, H))]
    e_spec = pl.BlockSpec((NP, BB, H2), lambda i: (0, i, 0))
    x_spec = pl.BlockSpec((BB, H), lambda i: (i, 0))
    p_spec = pl.BlockSpec((BB, NCOLOR), lambda i: (i, 0))
    csum_spec = pl.BlockSpec((NP, H2), lambda i: (0, 0))
    e_shape = jax.ShapeDtypeStruct((NP, NBIRD, H2), BF16)
    x_shape = jax.ShapeDtypeStruct((NBIRD, H), F32)
    csum_shape = jax.ShapeDtypeStruct((NP, H2), F32)

    e1, x1, csum = pl.pallas_call(
        _layer0_body,
        grid=(NBLK,),
        in_specs=[p_spec, _full((NCOLOR, H)), _full((1, H)),
                  _full((H, H2)), _full((NP, NCOLOR, H2)), _full((NP, H2)),
                  _full((H2, H2)), _full((H, H)), _full((H2, H)),
                  _full((1, H)), _full((H, H)), _full((1, H))],
        out_specs=[e_spec, x_spec, csum_spec],
        out_shape=[e_shape, x_shape, csum_shape],
    )(probs, wnT, bn_r, AA[0], su, base2(0, x_color), E2BD[0],
      N1a[0], N1bS[0], nb1f[0], N2[0], nb2_[0])
    x_color = color_update(0, x_color, csum)

    e2, x2, csum = pl.pallas_call(
        _mid_body,
        grid=(NBLK,),
        in_specs=[e_spec, x_spec] + wspecs,
        out_specs=[e_spec, x_spec, csum_spec],
        out_shape=[e_shape, x_shape, csum_shape],
    )(e1, x1, AA[1], CBD[1], base2(1, x_color), E2BD[1],
      N1a[1], N1bS[1], nb1f[1], N2[1], nb2_[1])
    x_color = color_update(1, x_color, csum)

    out = pl.pallas_call(
        _last_body,
        grid=(NBLK,),
        in_specs=[e_spec, x_spec, p_spec] + wspecs + [_full((H, NCOLOR)),
                                                      _full((1, NCOLOR))],
        out_specs=p_spec,
        out_shape=jax.ShapeDtypeStruct((NBIRD, NCOLOR), F32),
    )(e2, x2, probs, AA[2], CBD[2], base2(2, x_color), E2BD[2],
      N1a[2], N1bS[2], nb1f[2], N2[2], nb2_[2], f(Wc).T, f(bc)[None, :])
    return out


# speedup vs baseline: 2.1489x; 1.2366x over previous
import jax
import jax.numpy as jnp
from jax.experimental import pallas as pl

NBIRD = 50000
NCOLOR = 16
NP = NCOLOR // 2
H = 64
H2 = 2 * H
BB = 2000
NBLK = NBIRD // BB
F32 = jnp.float32
BF16 = jnp.bfloat16


def _dot(a, b):
    return jnp.dot(a, b, preferred_element_type=F32)


def _csum_accum(csum_ref, part):
    pid = pl.program_id(0)

    @pl.when(pid == 0)
    def _():
        csum_ref[:] = part

    @pl.when(pid > 0)
    def _():
        csum_ref[:] = csum_ref[:] + part


def _layer0_body(probs_ref, wnT_ref, bn_ref, aa_ref, su_ref, base2_ref,
                 e2bd_ref, n1aT_ref, n1bS_ref, nb1_ref, n2T_ref, nb2_ref,
                 e_out_ref, x_out_ref, csum_ref):
    p = probs_ref[:]
    xb = _dot(p, wnT_ref[:]) + bn_ref[:]
    ba2 = _dot(xb, aa_ref[:])
    pb = p.astype(BF16)
    e2bd = e2bd_ref[:]
    pu = jnp.concatenate([_dot(pb, su_ref[g]) for g in range(NP)],
                         axis=0).reshape(NP, BB, H2)
    pre = pu + ba2[None, :, :] + base2_ref[:][:, None, :]
    h = jnp.maximum(pre, 0.0).astype(BF16).reshape(NP * BB, H2)
    en = _dot(h, e2bd_ref[:]).reshape(NP, BB, H2)
    e_out_ref[:] = en.astype(e_out_ref.dtype)
    aggr2 = jnp.sum(en, axis=0)
    part = jnp.sum(en, axis=1)
    h2 = jnp.maximum(
        _dot(xb, n1aT_ref[:]) + _dot(aggr2, n1bS_ref[:]) + nb1_ref[:], 0.0)
    x_out_ref[:] = _dot(h2, n2T_ref[:]) + nb2_ref[:]
    _csum_accum(csum_ref, part)


def _mid_body(e_ref, x_ref, aa_ref, cbd_ref, base2_ref, e2bd_ref,
              n1aT_ref, n1bS_ref, nb1_ref, n2T_ref, nb2_ref,
              e_out_ref, x_out_ref, csum_ref):
    x = x_ref[:]
    ba2 = _dot(x, aa_ref[:])
    cbd = cbd_ref[:]
    e2bd = e2bd_ref[:]
    ec = _dot(e_ref[:].reshape(NP * BB, H2), cbd).reshape(NP, BB, H2)
    pre = ec + ba2[None, :, :] + base2_ref[:][:, None, :]
    h = jnp.maximum(pre, 0.0).astype(BF16).reshape(NP * BB, H2)
    en = _dot(h, e2bd).reshape(NP, BB, H2)
    e_out_ref[:] = en.astype(e_out_ref.dtype)
    aggr2 = jnp.sum(en, axis=0)
    part = jnp.sum(en, axis=1)
    h2 = jnp.maximum(
        _dot(x, n1aT_ref[:]) + _dot(aggr2, n1bS_ref[:]) + nb1_ref[:], 0.0)
    x_out_ref[:] = _dot(h2, n2T_ref[:]) + nb2_ref[:]
    _csum_accum(csum_ref, part)


def _last_body(e_ref, x_ref, probs_ref, aa_ref, cbd_ref, base2_ref,
               e2bd_ref, n1aT_ref, n1bS_ref, nb1_ref, n2T_ref, nb2_ref,
               wcT_ref, bc_ref, out_ref):
    x = x_ref[:]
    ba2 = _dot(x, aa_ref[:])
    cbd = cbd_ref[:]
    e2bd = e2bd_ref[:]
    ec = _dot(e_ref[:].reshape(NP * BB, H2), cbd).reshape(NP, BB, H2)
    pre = ec + ba2[None, :, :] + base2_ref[:][:, None, :]
    h = jnp.maximum(pre, 0.0).astype(BF16).reshape(NP * BB, H2)
    en = _dot(h, e2bd).reshape(NP, BB, H2)
    aggr2 = jnp.sum(en, axis=0)
    h2 = jnp.maximum(
        _dot(x, n1aT_ref[:]) + _dot(aggr2, n1bS_ref[:]) + nb1_ref[:], 0.0)
    xn = _dot(h2, n2T_ref[:]) + nb2_ref[:]
    scores = _dot(xn, wcT_ref[:]) + bc_ref[:]
    out_ref[:] = scores * probs_ref[:]


def _full(shape):
    return pl.BlockSpec(shape, lambda i: tuple(0 for _ in shape))


def kernel(probs, Wn, bn, We, be, eW1, eb1, eW2, eb2, nW1, nb1, nW2, nb2,
           Wc, bc):
    f = lambda a: a.astype(F32)
    probs = f(probs)
    wnT = f(Wn).T
    x_color = wnT + f(bn)[None, :]
    A = [f(eW1[l][:, :H]).T for l in range(3)]
    Bm = [f(eW1[l][:, H:2 * H]).T for l in range(3)]
    Cm = [f(eW1[l][:, 2 * H:]).T for l in range(3)]
    E2 = [f(eW2[l]).T for l in range(3)]
    N1a = [f(nW1[l][:, :H]).T for l in range(3)]
    N1b = [f(nW1[l][:, H:]).T for l in range(3)]
    N2 = [f(nW2[l]).T for l in range(3)]
    eb1_ = [f(eb1[l])[None, :] for l in range(3)]
    eb2_ = [f(eb2[l])[None, :] for l in range(3)]
    nb1_ = [f(nb1[l])[None, :] for l in range(3)]
    nb2_ = [f(nb2[l])[None, :] for l in range(3)]
    bn_r = f(bn)[None, :]
    u0 = (f(We)[:, 0] @ Cm[0])[None, :]
    v0 = (f(be) @ Cm[0])[None, :]

    def bd(w):
        z = jnp.zeros_like(w)
        top = jnp.concatenate([w, z], axis=1)
        bot = jnp.concatenate([z, w], axis=1)
        return jnp.concatenate([top, bot], axis=0)

    AA = [jnp.concatenate([A[l], A[l]], axis=1) for l in range(3)]
    CBD = [bd(Cm[l]).astype(BF16) for l in range(3)]
    E2BD = [bd(E2[l]).astype(BF16) for l in range(3)]
    N1bS = [jnp.concatenate([N1b[l], N1b[l]], axis=0) for l in range(3)]
    nb1f = [nb1_[l] + (NCOLOR * eb2_[l]) @ N1b[l] for l in range(3)]

    def base2(l, xc):
        b = xc @ Bm[l] + eb1_[l]
        if l == 0:
            b = b + v0
        else:
            b = b + eb2_[l - 1] @ Cm[l]
        return b.reshape(NP, H2)

    def color_update(l, xc, csum):
        aggr_c = csum.reshape(NCOLOR, H) + NBIRD * eb2_[l]
        h2 = jnp.maximum(xc @ N1a[l] + aggr_c @ N1b[l] + nb1_[l], 0.0)
        return h2 @ N2[l] + nb2_[l]

    eyeN = jnp.eye(NCOLOR, dtype=F32)
    su = jnp.stack([
        jnp.concatenate([eyeN[:, 2 * g:2 * g + 1] @ u0,
                         eyeN[:, 2 * g + 1:2 * g + 2] @ u0], axis=1)
        for g in range(NP)
    ]).astype(BF16)

    wspecs = [_full((H, H2)), _full((H2, H2)), _full((NP, H2)),
              _full((H2, H2)), _full((H, H)), _full((H2, H)),
              _full((1, H)), _full((H, H)), _full((1, H))]
    e_spec = pl.BlockSpec((NP, BB, H2), lambda i: (0, i, 0))
    x_spec = pl.BlockSpec((BB, H), lambda i: (i, 0))
    p_spec = pl.BlockSpec((BB, NCOLOR), lambda i: (i, 0))
    csum_spec = pl.BlockSpec((NP, H2), lambda i: (0, 0))
    e_shape = jax.ShapeDtypeStruct((NP, NBIRD, H2), BF16)
    x_shape = jax.ShapeDtypeStruct((NBIRD, H), F32)
    csum_shape = jax.ShapeDtypeStruct((NP, H2), F32)

    e1, x1, csum = pl.pallas_call(
        _layer0_body,
        grid=(NBLK,),
        in_specs=[p_spec, _full((NCOLOR, H)), _full((1, H)),
                  _full((H, H2)), _full((NP, NCOLOR, H2)), _full((NP, H2)),
                  _full((H2, H2)), _full((H, H)), _full((H2, H)),
                  _full((1, H)), _full((H, H)), _full((1, H))],
        out_specs=[e_spec, x_spec, csum_spec],
        out_shape=[e_shape, x_shape, csum_shape],
    )(probs, wnT, bn_r, AA[0], su, base2(0, x_color), E2BD[0],
      N1a[0], N1bS[0], nb1f[0], N2[0], nb2_[0])
    x_color = color_update(0, x_color, csum)

    e2, x2, csum = pl.pallas_call(
        _mid_body,
        grid=(NBLK,),
        in_specs=[e_spec, x_spec] + wspecs,
        out_specs=[e_spec, x_spec, csum_spec],
        out_shape=[e_shape, x_shape, csum_shape],
    )(e1, x1, AA[1], CBD[1], base2(1, x_color), E2BD[1],
      N1a[1], N1bS[1], nb1f[1], N2[1], nb2_[1])
    x_color = color_update(1, x_color, csum)

    out = pl.pallas_call(
        _last_body,
        grid=(NBLK,),
        in_specs=[e_spec, x_spec, p_spec] + wspecs + [_full((H, NCOLOR)),
                                                      _full((1, NCOLOR))],
        out_specs=p_spec,
        out_shape=jax.ShapeDtypeStruct((NBIRD, NCOLOR), F32),
    )(e2, x2, probs, AA[2], CBD[2], base2(2, x_color), E2BD[2],
      N1a[2], N1bS[2], nb1f[2], N2[2], nb2_[2], f(Wc).T, f(bc)[None, :])
    return out
